# baseline (device time: 31666 ns/iter reference)
import jax
import jax.numpy as jnp
from jax import lax
from jax.experimental import pallas as pl
from jax.experimental.pallas import tpu as pltpu

N_DEV = 32
A_BITS = (0, 1, 3)
B_BITS = (2, 4)


def kernel(x, Wg, Wu, Wd):
    m, _ = x.shape
    d_out = Wd.shape[1]
    seg = m // N_DEV

    def _seg_of(dev):
        qa = ((dev >> A_BITS[0]) & 1) | (((dev >> A_BITS[1]) & 1) << 1) | (
            ((dev >> A_BITS[2]) & 1) << 2
        )
        qb = ((dev >> B_BITS[0]) & 1) | (((dev >> B_BITS[1]) & 1) << 1)
        return qa * 4 + qb

    def body(
        x_ref,
        wg_ref,
        wu_ref,
        wd_ref,
        out_ref,
        comm_ref,
        recv_ref,
        rs_send,
        rs_recv,
        ag_send,
        ag_recv,
    ):
        my_id = lax.axis_index("i")

        barrier = pltpu.get_barrier_semaphore()
        for d in range(1, N_DEV):
            pl.semaphore_signal(
                barrier,
                inc=1,
                device_id=(my_id ^ d,),
                device_id_type=pl.DeviceIdType.MESH,
            )

        xb = x_ref[:].astype(jnp.bfloat16)
        gate = jnp.dot(
            xb, wg_ref[:].astype(jnp.bfloat16), preferred_element_type=jnp.float32
        )
        up = jnp.dot(
            xb, wu_ref[:].astype(jnp.bfloat16), preferred_element_type=jnp.float32
        )
        h = (gate * (up * jax.nn.sigmoid(up))).astype(jnp.bfloat16)
        comm_ref[:] = jnp.dot(
            h, wd_ref[:].astype(jnp.bfloat16), preferred_element_type=jnp.float32
        ).astype(jnp.bfloat16)

        pl.semaphore_wait(barrier, N_DEV - 1)

        my_lo = _seg_of(my_id) * seg

        rdmas = []
        for d in range(1, N_DEV):
            peer = my_id ^ d
            rdma = pltpu.make_async_remote_copy(
                src_ref=comm_ref.at[pl.ds(_seg_of(peer) * seg, seg), :],
                dst_ref=recv_ref.at[d - 1],
                send_sem=rs_send.at[d - 1],
                recv_sem=rs_recv.at[d - 1],
                device_id=(peer,),
                device_id_type=pl.DeviceIdType.MESH,
            )
            rdma.start()
            rdmas.append(rdma)
        for rdma in rdmas:
            rdma.wait()
        acc = comm_ref[pl.ds(my_lo, seg), :].astype(jnp.float32)
        for d in range(1, N_DEV):
            acc = acc + recv_ref[d - 1, :, :].astype(jnp.float32)
        comm_ref[pl.ds(my_lo, seg), :] = acc.astype(jnp.bfloat16)

        rdmas = []
        for d in range(1, N_DEV):
            rdma = pltpu.make_async_remote_copy(
                src_ref=comm_ref.at[pl.ds(my_lo, seg), :],
                dst_ref=comm_ref.at[pl.ds(my_lo, seg), :],
                send_sem=ag_send.at[d - 1],
                recv_sem=ag_recv.at[d - 1],
                device_id=(my_id ^ d,),
                device_id_type=pl.DeviceIdType.MESH,
            )
            rdma.start()
            rdmas.append(rdma)
        for rdma in rdmas:
            rdma.wait()

        out_ref[:] = comm_ref[:].astype(jnp.float32)

    return pl.pallas_call(
        body,
        out_shape=jax.ShapeDtypeStruct((m, d_out), jnp.float32),
        in_specs=[pl.BlockSpec(memory_space=pltpu.VMEM)] * 4,
        out_specs=pl.BlockSpec(memory_space=pltpu.VMEM),
        scratch_shapes=[
            pltpu.VMEM((m, d_out), jnp.bfloat16),
            pltpu.VMEM((N_DEV - 1, seg, d_out), jnp.bfloat16),
            pltpu.SemaphoreType.DMA((N_DEV - 1,)),
            pltpu.SemaphoreType.DMA((N_DEV - 1,)),
            pltpu.SemaphoreType.DMA((N_DEV - 1,)),
            pltpu.SemaphoreType.DMA((N_DEV - 1,)),
        ],
        compiler_params=pltpu.CompilerParams(collective_id=0),
    )(x, Wg, Wu, Wd)


# device time: 30253 ns/iter; 1.0467x vs baseline; 1.0467x over previous
import jax
import jax.numpy as jnp
from jax import lax
from jax.experimental import pallas as pl
from jax.experimental.pallas import tpu as pltpu

N_DEV = 32
A_BITS = (0, 1, 3)
B_BITS = (2, 4)


def _mask(d, bits):
    return sum(((d >> j) & 1) << b for j, b in enumerate(bits))


def kernel(x, Wg, Wu, Wd):
    m, _ = x.shape
    d_out = Wd.shape[1]
    seg_a = m // 8

    def body(
        x_ref,
        wg_ref,
        wu_ref,
        wd_ref,
        out_ref,
        comm_ref,
        recv_a,
        recv_b,
        rsa_send,
        rsa_recv,
        xb_send,
        xb_recv,
        aga_send,
        aga_recv,
    ):
        my_id = lax.axis_index("i")

        barrier = pltpu.get_barrier_semaphore()
        n_partners = 0
        for bits, radix in ((A_BITS, 8), (B_BITS, 4)):
            for d in range(1, radix):
                pl.semaphore_signal(
                    barrier,
                    inc=1,
                    device_id=(my_id ^ _mask(d, bits),),
                    device_id_type=pl.DeviceIdType.MESH,
                )
                n_partners += 1

        xb = x_ref[:].astype(jnp.bfloat16)
        gate = jnp.dot(
            xb, wg_ref[:].astype(jnp.bfloat16), preferred_element_type=jnp.float32
        )
        up = jnp.dot(
            xb, wu_ref[:].astype(jnp.bfloat16), preferred_element_type=jnp.float32
        )
        h = (gate * (up * jax.nn.sigmoid(up))).astype(jnp.bfloat16)
        comm_ref[:] = jnp.dot(
            h, wd_ref[:].astype(jnp.bfloat16), preferred_element_type=jnp.float32
        ).astype(jnp.bfloat16)

        pl.semaphore_wait(barrier, n_partners)

        qa = ((my_id >> A_BITS[0]) & 1) | (((my_id >> A_BITS[1]) & 1) << 1) | (
            ((my_id >> A_BITS[2]) & 1) << 2
        )
        lo = qa * seg_a
        rdmas = []
        for d in range(1, 8):
            pq = qa ^ d
            rdma = pltpu.make_async_remote_copy(
                src_ref=comm_ref.at[pl.ds(pq * seg_a, seg_a), :],
                dst_ref=recv_a.at[d - 1],
                send_sem=rsa_send.at[d - 1],
                recv_sem=rsa_recv.at[d - 1],
                device_id=(my_id ^ _mask(d, A_BITS),),
                device_id_type=pl.DeviceIdType.MESH,
            )
            rdma.start()
            rdmas.append(rdma)
        for rdma in rdmas:
            rdma.wait()
        acc = comm_ref[pl.ds(lo, seg_a), :].astype(jnp.float32)
        for d in range(1, 8):
            acc = acc + recv_a[d - 1, :, :].astype(jnp.float32)
        comm_ref[pl.ds(lo, seg_a), :] = acc.astype(jnp.bfloat16)

        rdmas = []
        for d in range(1, 4):
            rdma = pltpu.make_async_remote_copy(
                src_ref=comm_ref.at[pl.ds(lo, seg_a), :],
                dst_ref=recv_b.at[d - 1],
                send_sem=xb_send.at[d - 1],
                recv_sem=xb_recv.at[d - 1],
                device_id=(my_id ^ _mask(d, B_BITS),),
                device_id_type=pl.DeviceIdType.MESH,
            )
            rdma.start()
            rdmas.append(rdma)
        for rdma in rdmas:
            rdma.wait()
        acc = comm_ref[pl.ds(lo, seg_a), :].astype(jnp.float32)
        for d in range(1, 4):
            acc = acc + recv_b[d - 1, :, :].astype(jnp.float32)
        comm_ref[pl.ds(lo, seg_a), :] = acc.astype(jnp.bfloat16)

        rdmas = []
        for d in range(1, 8):
            rdma = pltpu.make_async_remote_copy(
                src_ref=comm_ref.at[pl.ds(lo, seg_a), :],
                dst_ref=comm_ref.at[pl.ds(lo, seg_a), :],
                send_sem=aga_send.at[d - 1],
                recv_sem=aga_recv.at[d - 1],
                device_id=(my_id ^ _mask(d, A_BITS),),
                device_id_type=pl.DeviceIdType.MESH,
            )
            rdma.start()
            rdmas.append(rdma)
        for rdma in rdmas:
            rdma.wait()

        out_ref[:] = comm_ref[:].astype(jnp.float32)

    return pl.pallas_call(
        body,
        out_shape=jax.ShapeDtypeStruct((m, d_out), jnp.float32),
        in_specs=[pl.BlockSpec(memory_space=pltpu.VMEM)] * 4,
        out_specs=pl.BlockSpec(memory_space=pltpu.VMEM),
        scratch_shapes=[
            pltpu.VMEM((m, d_out), jnp.bfloat16),
            pltpu.VMEM((7, seg_a, d_out), jnp.bfloat16),
            pltpu.VMEM((3, seg_a, d_out), jnp.bfloat16),
            pltpu.SemaphoreType.DMA((7,)),
            pltpu.SemaphoreType.DMA((7,)),
            pltpu.SemaphoreType.DMA((3,)),
            pltpu.SemaphoreType.DMA((3,)),
            pltpu.SemaphoreType.DMA((7,)),
            pltpu.SemaphoreType.DMA((7,)),
        ],
        compiler_params=pltpu.CompilerParams(collective_id=0),
    )(x, Wg, Wu, Wd)
